# baseline (speedup 1.0000x reference)
import functools

import jax
import jax.numpy as jnp
from jax import lax
from jax.experimental import pallas as pl
from jax.experimental.pallas import tpu as pltpu
from jax.experimental.pallas import tpu_sc as plsc

B = 16384
D = 32
NC = 2
NS = 16
NW = NC * NS
BPW = B // NW
CHUNK = 128
NCHUNK = BPW // CHUNK


def _body(idx_hbm, scal_hbm, table_hbm, params_hbm, out_hbm,
          idx_v, scal_v, params_v, rows_v, out_v, sem):
    wid = lax.axis_index("s") * NC + lax.axis_index("c")
    base = wid * BPW

    pltpu.sync_copy(idx_hbm.at[pl.ds(wid * NCHUNK, NCHUNK)], idx_v)
    pltpu.sync_copy(scal_hbm.at[wid], scal_v)
    pltpu.sync_copy(params_hbm, params_v)

    descs = [
        pltpu.async_copy(table_hbm.at[idx_v.at[j]],
                         rows_v.at[pl.ds(j * CHUNK, CHUNK)], sem)
        for j in range(NCHUNK)
    ]

    def half(off, h):
        return params_v[pl.ds(off + 16 * h, 16)]

    wt = [half(0, h) for h in range(2)]
    ww = [half(64, h) for h in range(2)]
    ws = [half(128, h) for h in range(2)]
    p0 = [half(192, h) for h in range(2)]
    c1 = [half(32, h) + half(224, h) for h in range(2)]
    c2 = [half(96, h) + half(256, h) for h in range(2)]
    c3 = [half(160, h) + half(288, h) for h in range(2)]

    def dense_chunk(c, carry):
        r0 = c * 16
        tv = scal_v[pl.ds(r0, 16)]
        wv = scal_v[pl.ds(r0 + BPW, 16)]
        sv = scal_v[pl.ds(r0 + 2 * BPW, 16)]
        for i in range(16):
            r = r0 + i
            t = jnp.full((16,), tv[i], jnp.float32)
            wx = jnp.full((16,), wv[i], jnp.float32)
            sp = jnp.full((16,), sv[i], jnp.float32)
            for h in range(2):
                sl = pl.ds(16 * h, 16)
                out_v[r, 1, sl] = t * wt[h] + c1[h]
                out_v[r, 2, sl] = wx * ww[h] + c2[h]
                out_v[r, 3, sl] = sp * ws[h] + c3[h]
        return carry

    lax.fori_loop(0, BPW // 16, dense_chunk, 0)

    for d in descs:
        d.wait()

    def station_row(r, carry):
        for h in range(2):
            sl = pl.ds(16 * h, 16)
            out_v[r, 0, sl] = rows_v[r, sl] + p0[h]
        return carry

    lax.fori_loop(0, BPW, station_row, 0)

    pltpu.sync_copy(out_v, out_hbm.at[pl.ds(base, BPW)])


@jax.jit
def _tokenize(idx2d, scal, table, params):
    mesh = plsc.VectorSubcoreMesh(core_axis_name="c", subcore_axis_name="s")
    return pl.kernel(
        _body,
        out_type=jax.ShapeDtypeStruct((B, 4, D), jnp.float32),
        mesh=mesh,
        compiler_params=pltpu.CompilerParams(use_tc_tiling_on_sc=False),
        scratch_types=[
            pltpu.VMEM((NCHUNK, CHUNK), jnp.int32),
            pltpu.VMEM((3 * BPW,), jnp.float32),
            pltpu.VMEM((320,), jnp.float32),
            pltpu.VMEM((BPW, D), jnp.float32),
            pltpu.VMEM((BPW, 4, D), jnp.float32),
            pltpu.SemaphoreType.DMA,
        ],
    )(idx2d, scal, table, params)


def kernel(station_ids, time_of_day, weather_index, sports_event,
           station_emb, w_time, b_time, w_weather, b_weather,
           w_sports, b_sports, pos_emb):
    idx2d = station_ids.astype(jnp.int32).reshape(B // CHUNK, CHUNK)
    scal = jnp.stack([
        time_of_day.astype(jnp.float32).reshape(NW, BPW),
        weather_index.astype(jnp.float32).reshape(NW, BPW),
        sports_event.astype(jnp.float32).reshape(NW, BPW),
    ], axis=1).reshape(NW, 3 * BPW)
    params = jnp.concatenate([
        w_time, b_time, w_weather, b_weather, w_sports, b_sports,
        pos_emb.reshape(-1),
    ])
    return _tokenize(idx2d, scal, station_emb, params)


# speedup vs baseline: 1.0584x; 1.0584x over previous
import functools

import jax
import jax.numpy as jnp
from jax import lax
from jax.experimental import pallas as pl
from jax.experimental.pallas import tpu as pltpu
from jax.experimental.pallas import tpu_sc as plsc

B = 16384
D = 32
NC = 2
NS = 16
NW = NC * NS
BPW = B // NW
CHUNK = 128
NCHUNK = BPW // CHUNK


def _body(idx_hbm, scal_hbm, table_hbm, params_hbm, out_hbm,
          idx_v, scal_v, params_v, rows_v, out_v, sem):
    wid = lax.axis_index("s") * NC + lax.axis_index("c")
    base = wid * BPW

    pltpu.sync_copy(idx_hbm.at[pl.ds(wid * NCHUNK, NCHUNK)], idx_v)
    pltpu.sync_copy(scal_hbm.at[pl.ds(wid * 3 * BPW, 3 * BPW)], scal_v)
    pltpu.sync_copy(params_hbm, params_v)

    descs = [
        pltpu.async_copy(table_hbm.at[idx_v.at[j]],
                         rows_v.at[pl.ds(j * CHUNK, CHUNK)], sem)
        for j in range(NCHUNK)
    ]

    def half(off, h):
        return params_v[pl.ds(off + 16 * h, 16)]

    wt = [half(0, h) for h in range(2)]
    ww = [half(64, h) for h in range(2)]
    ws = [half(128, h) for h in range(2)]
    p0 = [half(192, h) for h in range(2)]
    c1 = [half(32, h) + half(224, h) for h in range(2)]
    c2 = [half(96, h) + half(256, h) for h in range(2)]
    c3 = [half(160, h) + half(288, h) for h in range(2)]

    def dense_chunk(c, carry):
        r0 = c * 16
        tv = scal_v[pl.ds(r0, 16)]
        wv = scal_v[pl.ds(r0 + BPW, 16)]
        sv = scal_v[pl.ds(r0 + 2 * BPW, 16)]
        for i in range(16):
            r = r0 + i
            t = jnp.full((16,), tv[i], jnp.float32)
            wx = jnp.full((16,), wv[i], jnp.float32)
            sp = jnp.full((16,), sv[i], jnp.float32)
            for h in range(2):
                sl = pl.ds(16 * h, 16)
                out_v[r, pl.ds(D + 16 * h, 16)] = t * wt[h] + c1[h]
                out_v[r, pl.ds(2 * D + 16 * h, 16)] = wx * ww[h] + c2[h]
                out_v[r, pl.ds(3 * D + 16 * h, 16)] = sp * ws[h] + c3[h]
        return carry

    lax.fori_loop(0, BPW // 16, dense_chunk, 0)

    for d in descs:
        d.wait()

    def station_row(r, carry):
        for h in range(2):
            sl = pl.ds(16 * h, 16)
            out_v[r, pl.ds(16 * h, 16)] = rows_v[r, sl] + p0[h]
        return carry

    lax.fori_loop(0, BPW, station_row, 0)

    pltpu.sync_copy(out_v, out_hbm.at[pl.ds(base, BPW)])


@jax.jit
def _tokenize(idx2d, scal, table, params):
    mesh = plsc.VectorSubcoreMesh(core_axis_name="c", subcore_axis_name="s")
    return pl.kernel(
        _body,
        out_type=jax.ShapeDtypeStruct((B, 4 * D), jnp.float32),
        mesh=mesh,
        compiler_params=pltpu.CompilerParams(use_tc_tiling_on_sc=False),
        scratch_types=[
            pltpu.VMEM((NCHUNK, CHUNK), jnp.int32),
            pltpu.VMEM((3 * BPW,), jnp.float32),
            pltpu.VMEM((320,), jnp.float32),
            pltpu.VMEM((BPW, D), jnp.float32),
            pltpu.VMEM((BPW, 4 * D), jnp.float32),
            pltpu.SemaphoreType.DMA,
        ],
    )(idx2d, scal, table, params)


def kernel(station_ids, time_of_day, weather_index, sports_event,
           station_emb, w_time, b_time, w_weather, b_weather,
           w_sports, b_sports, pos_emb):
    idx2d = station_ids.astype(jnp.int32).reshape(B // CHUNK, CHUNK)
    scal = jnp.stack([
        time_of_day.astype(jnp.float32).reshape(NW, BPW),
        weather_index.astype(jnp.float32).reshape(NW, BPW),
        sports_event.astype(jnp.float32).reshape(NW, BPW),
    ], axis=1).reshape(-1)
    params = jnp.concatenate([
        w_time, b_time, w_weather, b_weather, w_sports, b_sports,
        pos_emb.reshape(-1),
    ])
    out = _tokenize(idx2d, scal, station_emb, params)
    return out.reshape(B, 4, D)
